# Initial kernel scaffold; baseline (speedup 1.0000x reference)
import jax
import jax.numpy as jnp
from jax.experimental import pallas as pl


def kernel(f0, f1, f2, f3, f4, f5, f6, f7, f8, f9, f10, f11, f12, f13, f14, f15, f16, f17, f18, f19, f20, f21, f22, f23, f24, f25, t0, t1, t2, t3, t4, t5, t6, t7, t8, t9, t10, t11, t12, t13, t14, t15, t16, t17, t18, t19, t20, t21, t22, t23, t24, t25, W1, b1, W2, b2):
    raise NotImplementedError("write your pallas kernel here")


# speedup vs baseline: 2.0117x; 2.0117x over previous
import functools

import jax
import jax.numpy as jnp
from jax import lax
from jax.experimental import pallas as pl
from jax.experimental.pallas import tpu as pltpu
from jax.experimental.pallas import tpu_sc as plsc

NUM_FIELDS = 26
VOCAB = 100000
EMB = 32
BATCH = 16384
TOTAL_DIM = NUM_FIELDS * EMB
HID = 128

NC = 2
NS = 16
NW = NC * NS
B_PER_W = BATCH // NW
CHUNK = 128
NCHUNK = B_PER_W // CHUNK

@functools.cache
def _make_gather_sc():
    mesh = plsc.VectorSubcoreMesh(core_axis_name="c", subcore_axis_name="s")

    @functools.partial(
        pl.kernel,
        mesh=mesh,
        out_type=jax.ShapeDtypeStruct((BATCH, TOTAL_DIM), jnp.float32),
        scratch_types=[
            pltpu.VMEM((NUM_FIELDS, NCHUNK, CHUNK), jnp.int32),
            pltpu.VMEM((B_PER_W, EMB), jnp.float32),
            pltpu.VMEM((B_PER_W, EMB), jnp.float32),
            pltpu.SemaphoreType.DMA,
            pltpu.SemaphoreType.DMA,
        ],
        compiler_params=pltpu.CompilerParams(use_tc_tiling_on_sc=False),
    )
    def _gather_sc(idx_hbm, *rest):
        tables = rest[:NUM_FIELDS]
        out_hbm = rest[NUM_FIELDS]
        idx_v, rows_a, rows_b, sem_a, sem_b = rest[NUM_FIELDS + 1:]

        wid = lax.axis_index("s") * NC + lax.axis_index("c")
        base = wid * B_PER_W

        pltpu.sync_copy(idx_hbm.at[:, pl.ds(wid * NCHUNK, NCHUNK), :], idx_v)

        bufs = (rows_a, rows_b)
        sems = (sem_a, sem_b)

        def fire(i):
            rows, sem = bufs[i % 2], sems[i % 2]
            return [
                pltpu.async_copy(
                    tables[i].at[idx_v.at[i, c]],
                    rows.at[pl.ds(c * CHUNK, CHUNK)],
                    sem,
                )
                for c in range(NCHUNK)
            ]

        pending = fire(0)
        for i in range(NUM_FIELDS):
            descs = pending
            if i + 1 < NUM_FIELDS:
                pending = fire(i + 1)
            for d in descs:
                d.wait()
            pltpu.sync_copy(
                bufs[i % 2],
                out_hbm.at[pl.ds(base, B_PER_W), pl.ds(i * EMB, EMB)],
            )

    return _gather_sc


BLK = 2048


def _mlp_body(x_ref, w1_ref, b1_ref, w2m_ref, w2l_ref, b2m_ref, b2l_ref,
              eps_ref, z_ref, mu_ref, lv_ref):
    x = x_ref[...]
    h = jnp.maximum(
        jax.lax.dot_general(x, w1_ref[...], (((1,), (0,)), ((), ())),
                            preferred_element_type=jnp.float32) + b1_ref[...],
        0.0,
    )
    mu = jax.lax.dot_general(h, w2m_ref[...], (((1,), (0,)), ((), ())),
                             preferred_element_type=jnp.float32) + b2m_ref[...]
    lv = jax.lax.dot_general(h, w2l_ref[...], (((1,), (0,)), ((), ())),
                             preferred_element_type=jnp.float32) + b2l_ref[...]
    std = jnp.exp(0.5 * lv)
    z_ref[...] = mu + eps_ref[...] * std
    mu_ref[...] = mu
    lv_ref[...] = lv


def _mlp_tc(cond, w1, b1, w2m, w2l, b2m, b2l, eps):
    nblk = BATCH // BLK
    out_sd = jax.ShapeDtypeStruct((BATCH, EMB), jnp.float32)
    return pl.pallas_call(
        _mlp_body,
        grid=(nblk,),
        in_specs=[
            pl.BlockSpec((BLK, TOTAL_DIM), lambda i: (i, 0)),
            pl.BlockSpec((TOTAL_DIM, HID), lambda i: (0, 0)),
            pl.BlockSpec((1, HID), lambda i: (0, 0)),
            pl.BlockSpec((HID, EMB), lambda i: (0, 0)),
            pl.BlockSpec((HID, EMB), lambda i: (0, 0)),
            pl.BlockSpec((1, EMB), lambda i: (0, 0)),
            pl.BlockSpec((1, EMB), lambda i: (0, 0)),
            pl.BlockSpec((BLK, EMB), lambda i: (i, 0)),
        ],
        out_specs=[
            pl.BlockSpec((BLK, EMB), lambda i: (i, 0)),
            pl.BlockSpec((BLK, EMB), lambda i: (i, 0)),
            pl.BlockSpec((BLK, EMB), lambda i: (i, 0)),
        ],
        out_shape=[out_sd, out_sd, out_sd],
        compiler_params=pltpu.CompilerParams(
            dimension_semantics=("parallel",),
        ),
    )(cond, w1, b1, w2m, w2l, b2m, b2l, eps)


def kernel(f0, f1, f2, f3, f4, f5, f6, f7, f8, f9, f10, f11, f12, f13, f14,
           f15, f16, f17, f18, f19, f20, f21, f22, f23, f24, f25,
           t0, t1, t2, t3, t4, t5, t6, t7, t8, t9, t10, t11, t12, t13, t14,
           t15, t16, t17, t18, t19, t20, t21, t22, t23, t24, t25,
           W1, b1, W2, b2):
    fs = (f0, f1, f2, f3, f4, f5, f6, f7, f8, f9, f10, f11, f12, f13, f14,
          f15, f16, f17, f18, f19, f20, f21, f22, f23, f24, f25)
    ts = (t0, t1, t2, t3, t4, t5, t6, t7, t8, t9, t10, t11, t12, t13, t14,
          t15, t16, t17, t18, t19, t20, t21, t22, t23, t24, t25)

    idx = jnp.stack(fs).reshape(NUM_FIELDS, BATCH // CHUNK, CHUNK)
    cond = _make_gather_sc()(idx, *ts)

    eps = jax.random.normal(jax.random.key(42), (BATCH, EMB), jnp.float32)
    z, mu, lv = _mlp_tc(
        cond, W1, b1.reshape(1, HID),
        W2[:, :EMB], W2[:, EMB:],
        b2[:EMB].reshape(1, EMB), b2[EMB:].reshape(1, EMB),
        eps,
    )
    return (z, mu, lv)
